# Initial kernel scaffold; baseline (speedup 1.0000x reference)
import jax
import jax.numpy as jnp
from jax.experimental import pallas as pl


def kernel(coords, offset_inst, offset_tree, tree_labels):
    raise NotImplementedError("write your pallas kernel here")


# speedup vs baseline: 2.5638x; 2.5638x over previous
import functools

import jax
import jax.numpy as jnp
from jax import lax
from jax.experimental import pallas as pl
from jax.experimental.pallas import tpu as pltpu
from jax.experimental.pallas import tpu_sc as plsc

N = 6400000
K = 1000
KP = 1024
NC = 2
NS = 16
NW = NC * NS
P = N // NW
T = 4000
NT = P // T
SUB = T // 16
LANES = 16
TBL = LANES * 4 * KP


def _sc_segment_sums(oi_flat, ot_flat, labels):
    mesh = plsc.VectorSubcoreMesh(core_axis_name="c", subcore_axis_name="s")

    @functools.partial(
        pl.kernel,
        mesh=mesh,
        out_type=jax.ShapeDtypeStruct((NW * 4, KP), jnp.float32),
        scratch_types=[
            pltpu.VMEM((TBL,), jnp.float32),
            pltpu.VMEM((4, KP), jnp.float32),
            pltpu.VMEM((T,), jnp.int32),
            pltpu.VMEM((3 * T,), jnp.float32),
            pltpu.VMEM((3 * T,), jnp.float32),
        ],
        compiler_params=pltpu.CompilerParams(needs_layout_passes=False),
    )
    def k(oi_hbm, ot_hbm, lab_hbm, out_hbm, table, res, lab_v, oi_v, ot_v):
        wid = lax.axis_index("s") * NC + lax.axis_index("c")
        lane = lax.iota(jnp.int32, 16)
        ivec_lab = lane * SUB
        ivec_f = lane * (3 * SUB)
        lanebase = lane * (4 * KP)
        zeros = jnp.zeros((16,), jnp.float32)
        ones = jnp.ones((16,), jnp.float32)

        def zero_body(i, _):
            table[pl.ds(i * 16, 16)] = zeros
            return 0

        lax.fori_loop(0, TBL // 16, zero_body, 0)

        pbase = wid * P

        for t in range(NT):
            pltpu.sync_copy(lab_hbm.at[pl.ds(pbase + t * T, T)], lab_v)
            pltpu.sync_copy(oi_hbm.at[pl.ds(3 * (pbase + t * T), 3 * T)], oi_v)
            pltpu.sync_copy(ot_hbm.at[pl.ds(3 * (pbase + t * T), 3 * T)], ot_v)

            def step(s, _):
                lab = plsc.load_gather(lab_v, [ivec_lab + s])
                fb = ivec_f + 3 * s
                dx = plsc.load_gather(oi_v, [fb]) - plsc.load_gather(ot_v, [fb])
                dy = plsc.load_gather(oi_v, [fb + 1]) - plsc.load_gather(ot_v, [fb + 1])
                dz = plsc.load_gather(oi_v, [fb + 2]) - plsc.load_gather(ot_v, [fb + 2])
                bi = lanebase + lab
                plsc.addupdate_scatter(table, [bi], dx)
                plsc.addupdate_scatter(table, [bi + KP], dy)
                plsc.addupdate_scatter(table, [bi + 2 * KP], dz)
                plsc.addupdate_scatter(table, [bi + 3 * KP], ones)
                return 0

            lax.fori_loop(0, SUB, step, 0)

        for c in range(4):
            def red_body(j, _, c=c):
                acc = zeros
                for l in range(LANES):
                    acc = acc + table[pl.ds(l * (4 * KP) + c * KP + j * 16, 16)]
                res[c, pl.ds(j * 16, 16)] = acc
                return 0

            lax.fori_loop(0, KP // 16, red_body, 0)

        pltpu.sync_copy(res, out_hbm.at[pl.ds(wid * 4, 4)])

    return k(oi_flat, ot_flat, labels)


def _loss_body(p_ref, o_ref):
    x = p_ref[...]
    rid = lax.broadcasted_iota(jnp.int32, (NW * 4, KP), 0)
    rmod = lax.rem(rid, 4)

    def csum(c):
        return jnp.sum(jnp.where(rmod == c, x, 0.0), axis=0, keepdims=True)

    s0, s1, s2, cnt = csum(0), csum(1), csum(2), csum(3)
    safe = jnp.where(cnt > 0.0, cnt, 1.0)
    d0, d1, d2 = s0 / safe, s1 / safe, s2 / safe
    pt = d0 * d0 + d1 * d1 + d2 * d2
    kidx = lax.broadcasted_iota(jnp.int32, (1, KP), 1)
    valid = kidx > 0
    contrib = (cnt >= 2.0) & valid
    present = (cnt >= 1.0) & valid
    total = jnp.sum(jnp.where(contrib, pt, 0.0))
    ntree = jnp.sum(jnp.where(present, 1.0, 0.0))
    loss = jnp.where(ntree > 0.0, total / jnp.maximum(ntree, 1.0), 0.0)
    o_ref[...] = jnp.full((1, 1), loss, jnp.float32)


def kernel(coords, offset_inst, offset_tree, tree_labels):
    del coords
    oi = offset_inst.reshape(-1)
    ot = offset_tree.reshape(-1)
    partials = _sc_segment_sums(oi, ot, tree_labels)
    loss = pl.pallas_call(
        _loss_body,
        out_shape=jax.ShapeDtypeStruct((1, 1), jnp.float32),
    )(partials)
    return jnp.reshape(loss, ())


# speedup vs baseline: 104.8563x; 40.8983x over previous
import functools

import jax
import jax.numpy as jnp
from jax import lax
from jax.experimental import pallas as pl
from jax.experimental.pallas import tpu as pltpu
from jax.experimental.pallas import tpu_sc as plsc

N = 6400000
K = 1000
KP = 1024
NC = 2
NS = 16
NW = NC * NS
P = N // NW
T = 4000
NT = P // T
SUB = T // 16
LANES = 16
TBL = LANES * 4 * KP


def _sc_segment_sums(dx, dy, dz, labels):
    mesh = plsc.VectorSubcoreMesh(core_axis_name="c", subcore_axis_name="s")

    @functools.partial(
        pl.kernel,
        mesh=mesh,
        out_type=jax.ShapeDtypeStruct((NW * 4, KP), jnp.float32),
        scratch_types=[
            pltpu.VMEM((TBL,), jnp.float32),
            pltpu.VMEM((4, KP), jnp.float32),
            pltpu.VMEM((T,), jnp.int32),
            pltpu.VMEM((T,), jnp.float32),
            pltpu.VMEM((T,), jnp.float32),
            pltpu.VMEM((T,), jnp.float32),
            pltpu.VMEM((T,), jnp.int32),
            pltpu.VMEM((T,), jnp.float32),
            pltpu.VMEM((T,), jnp.float32),
            pltpu.VMEM((T,), jnp.float32),
            pltpu.SemaphoreType.DMA,
            pltpu.SemaphoreType.DMA,
        ],
        compiler_params=pltpu.CompilerParams(needs_layout_passes=False),
    )
    def k(dx_hbm, dy_hbm, dz_hbm, lab_hbm, out_hbm,
          table, res, lab_v0, dx_v0, dy_v0, dz_v0,
          lab_v1, dx_v1, dy_v1, dz_v1, sem0, sem1):
        wid = lax.axis_index("s") * NC + lax.axis_index("c")
        lane = lax.iota(jnp.int32, 16)
        ivec = lane * SUB
        lanebase = lane * (4 * KP)
        zeros = jnp.zeros((16,), jnp.float32)
        ones = jnp.ones((16,), jnp.float32)
        izeros = jnp.zeros((16,), jnp.int32)
        bufs = ((lab_v0, dx_v0, dy_v0, dz_v0), (lab_v1, dx_v1, dy_v1, dz_v1))
        sems = (sem0, sem1)

        pbase = wid * P

        def start(t, b):
            off = pl.ds(pbase + t * T, T)
            lab_b, dx_b, dy_b, dz_b = bufs[b]
            sem = sems[b]
            return (
                pltpu.async_copy(lab_hbm.at[off], lab_b, sem),
                pltpu.async_copy(dx_hbm.at[off], dx_b, sem),
                pltpu.async_copy(dy_hbm.at[off], dy_b, sem),
                pltpu.async_copy(dz_hbm.at[off], dz_b, sem),
            )

        cur = (izeros, zeros, zeros, zeros, zeros)
        pending = start(0, 0)

        def zero_body(i, _):
            table[pl.ds(i * 16, 16)] = zeros
            return 0

        lax.fori_loop(0, TBL // 16, zero_body, 0)

        for t in range(NT):
            b = t % 2
            if t + 1 < NT:
                nxt = start(t + 1, 1 - b)
            else:
                nxt = None
            for h in pending:
                h.wait()
            pending = nxt

            lab_b, dx_b, dy_b, dz_b = bufs[b]

            def step(s, carry, lab_b=lab_b, dx_b=dx_b, dy_b=dy_b, dz_b=dz_b):
                clab, ax, ay, az, ac = carry
                pidx = ivec + s
                nlab = plsc.load_gather(lab_b, [pidx])
                vx = plsc.load_gather(dx_b, [pidx])
                vy = plsc.load_gather(dy_b, [pidx])
                vz = plsc.load_gather(dz_b, [pidx])
                flush = nlab != clab
                bi = lanebase + clab
                plsc.addupdate_scatter(table, [bi], ax, mask=flush)
                plsc.addupdate_scatter(table, [bi + KP], ay, mask=flush)
                plsc.addupdate_scatter(table, [bi + 2 * KP], az, mask=flush)
                plsc.addupdate_scatter(table, [bi + 3 * KP], ac, mask=flush)
                keep = jnp.where(flush, 0.0, 1.0)
                ax = ax * keep + vx
                ay = ay * keep + vy
                az = az * keep + vz
                ac = ac * keep + 1.0
                return (nlab, ax, ay, az, ac)

            cur = lax.fori_loop(0, SUB, step, cur)

        clab, ax, ay, az, ac = cur
        bi = lanebase + clab
        plsc.addupdate_scatter(table, [bi], ax)
        plsc.addupdate_scatter(table, [bi + KP], ay)
        plsc.addupdate_scatter(table, [bi + 2 * KP], az)
        plsc.addupdate_scatter(table, [bi + 3 * KP], ac)

        for c in range(4):
            def red_body(j, _, c=c):
                acc = zeros
                for l in range(LANES):
                    acc = acc + table[pl.ds(l * (4 * KP) + c * KP + j * 16, 16)]
                res[c, pl.ds(j * 16, 16)] = acc
                return 0

            lax.fori_loop(0, KP // 16, red_body, 0)

        pltpu.sync_copy(res, out_hbm.at[pl.ds(wid * 4, 4)])

    return k(dx, dy, dz, labels)


def _loss_body(p_ref, o_ref):
    x = p_ref[...]
    rid = lax.broadcasted_iota(jnp.int32, (NW * 4, KP), 0)
    rmod = lax.rem(rid, 4)

    def csum(c):
        return jnp.sum(jnp.where(rmod == c, x, 0.0), axis=0, keepdims=True)

    s0, s1, s2, cnt = csum(0), csum(1), csum(2), csum(3)
    safe = jnp.where(cnt > 0.0, cnt, 1.0)
    d0, d1, d2 = s0 / safe, s1 / safe, s2 / safe
    pt = d0 * d0 + d1 * d1 + d2 * d2
    kidx = lax.broadcasted_iota(jnp.int32, (1, KP), 1)
    valid = kidx > 0
    contrib = (cnt >= 2.0) & valid
    present = (cnt >= 1.0) & valid
    total = jnp.sum(jnp.where(contrib, pt, 0.0))
    ntree = jnp.sum(jnp.where(present, 1.0, 0.0))
    loss = jnp.where(ntree > 0.0, total / jnp.maximum(ntree, 1.0), 0.0)
    o_ref[...] = jnp.full((1, 1), loss, jnp.float32)


def kernel(coords, offset_inst, offset_tree, tree_labels):
    del coords
    dx = offset_inst[:, 0] - offset_tree[:, 0]
    dy = offset_inst[:, 1] - offset_tree[:, 1]
    dz = offset_inst[:, 2] - offset_tree[:, 2]
    partials = _sc_segment_sums(dx, dy, dz, tree_labels)
    loss = pl.pallas_call(
        _loss_body,
        out_shape=jax.ShapeDtypeStruct((1, 1), jnp.float32),
    )(partials)
    return jnp.reshape(loss, ())


# speedup vs baseline: 110.0257x; 1.0493x over previous
import functools

import jax
import jax.numpy as jnp
from jax import lax
from jax.experimental import pallas as pl
from jax.experimental.pallas import tpu as pltpu
from jax.experimental.pallas import tpu_sc as plsc

N = 6400000
K = 1000
KP = 1024
NC = 2
NS = 16
NW = NC * NS
P = N // NW
T = 4000
NT = P // T
SUB = T // 16
LANES = 16
TBL = LANES * 4 * KP


def _sc_segment_sums(dx, dy, dz, labels):
    mesh = plsc.VectorSubcoreMesh(core_axis_name="c", subcore_axis_name="s")

    @functools.partial(
        pl.kernel,
        mesh=mesh,
        out_type=jax.ShapeDtypeStruct((NW * 4, KP), jnp.float32),
        scratch_types=[
            pltpu.VMEM((TBL,), jnp.float32),
            pltpu.VMEM((4, KP), jnp.float32),
            pltpu.VMEM((T,), jnp.int32),
            pltpu.VMEM((T,), jnp.float32),
            pltpu.VMEM((T,), jnp.float32),
            pltpu.VMEM((T,), jnp.float32),
            pltpu.VMEM((T,), jnp.int32),
            pltpu.VMEM((T,), jnp.float32),
            pltpu.VMEM((T,), jnp.float32),
            pltpu.VMEM((T,), jnp.float32),
            pltpu.SemaphoreType.DMA,
            pltpu.SemaphoreType.DMA,
        ],
        compiler_params=pltpu.CompilerParams(needs_layout_passes=False),
    )
    def k(dx_hbm, dy_hbm, dz_hbm, lab_hbm, out_hbm,
          table, res, lab_v0, dx_v0, dy_v0, dz_v0,
          lab_v1, dx_v1, dy_v1, dz_v1, sem0, sem1):
        wid = lax.axis_index("s") * NC + lax.axis_index("c")
        lane = lax.iota(jnp.int32, 16)
        ivec = lane * SUB
        lanebase = lane * (4 * KP)
        zeros = jnp.zeros((16,), jnp.float32)
        ones = jnp.ones((16,), jnp.float32)
        izeros = jnp.zeros((16,), jnp.int32)
        bufs = ((lab_v0, dx_v0, dy_v0, dz_v0), (lab_v1, dx_v1, dy_v1, dz_v1))
        sems = (sem0, sem1)

        pbase = wid * P

        def start(t, b):
            off = pl.ds(pbase + t * T, T)
            lab_b, dx_b, dy_b, dz_b = bufs[b]
            sem = sems[b]
            pltpu.async_copy(lab_hbm.at[off], lab_b, sem)
            pltpu.async_copy(dx_hbm.at[off], dx_b, sem)
            pltpu.async_copy(dy_hbm.at[off], dy_b, sem)
            pltpu.async_copy(dz_hbm.at[off], dz_b, sem)

        cur = (izeros, zeros, zeros, zeros, zeros)
        start(0, 0)

        def zero_body(i, _):
            for u in range(8):
                table[pl.ds((i * 8 + u) * 16, 16)] = zeros
            return 0

        lax.fori_loop(0, TBL // 128, zero_body, 0)
        start(1, 1)

        UF = 5

        def make_step(lab_b, dx_b, dy_b, dz_b):
            def step(s, carry):
                clab, ax, ay, az, ac = carry
                pidx = ivec + s
                nlab = plsc.load_gather(lab_b, [pidx])
                vx = plsc.load_gather(dx_b, [pidx])
                vy = plsc.load_gather(dy_b, [pidx])
                vz = plsc.load_gather(dz_b, [pidx])
                flush = nlab != clab
                bi = lanebase + clab
                plsc.addupdate_scatter(table, [bi], ax, mask=flush)
                plsc.addupdate_scatter(table, [bi + KP], ay, mask=flush)
                plsc.addupdate_scatter(table, [bi + 2 * KP], az, mask=flush)
                plsc.addupdate_scatter(table, [bi + 3 * KP], ac, mask=flush)
                ax = jnp.where(flush, vx, ax + vx)
                ay = jnp.where(flush, vy, ay + vy)
                az = jnp.where(flush, vz, az + vz)
                ac = jnp.where(flush, 1.0, ac + 1.0)
                return (nlab, ax, ay, az, ac)

            def stepu(i, carry):
                for u in range(UF):
                    carry = step(i * UF + u, carry)
                return carry

            return stepu

        def pair(i, carry):
            for b in range(2):
                t = 2 * i + b
                lab_b, dx_b, dy_b, dz_b = bufs[b]
                sem = sems[b]
                pltpu.make_async_copy(lab_hbm.at[pl.ds(0, T)], lab_b, sem).wait()
                pltpu.make_async_copy(dx_hbm.at[pl.ds(0, T)], dx_b, sem).wait()
                pltpu.make_async_copy(dy_hbm.at[pl.ds(0, T)], dy_b, sem).wait()
                pltpu.make_async_copy(dz_hbm.at[pl.ds(0, T)], dz_b, sem).wait()
                carry = lax.fori_loop(
                    0, SUB // UF, make_step(lab_b, dx_b, dy_b, dz_b), carry
                )

                @pl.when(t + 2 < NT)
                def _():
                    start(t + 2, b)

            return carry

        cur = lax.fori_loop(0, NT // 2, pair, cur)

        clab, ax, ay, az, ac = cur
        bi = lanebase + clab
        plsc.addupdate_scatter(table, [bi], ax)
        plsc.addupdate_scatter(table, [bi + KP], ay)
        plsc.addupdate_scatter(table, [bi + 2 * KP], az)
        plsc.addupdate_scatter(table, [bi + 3 * KP], ac)

        for c in range(4):
            def red_body(j, _, c=c):
                acc = zeros
                for l in range(LANES):
                    acc = acc + table[pl.ds(l * (4 * KP) + c * KP + j * 16, 16)]
                res[c, pl.ds(j * 16, 16)] = acc
                return 0

            lax.fori_loop(0, KP // 16, red_body, 0)

        pltpu.sync_copy(res, out_hbm.at[pl.ds(wid * 4, 4)])

    return k(dx, dy, dz, labels)


def _loss_body(p_ref, o_ref):
    x = p_ref[...]
    rid = lax.broadcasted_iota(jnp.int32, (NW * 4, KP), 0)
    rmod = lax.rem(rid, 4)

    def csum(c):
        return jnp.sum(jnp.where(rmod == c, x, 0.0), axis=0, keepdims=True)

    s0, s1, s2, cnt = csum(0), csum(1), csum(2), csum(3)
    safe = jnp.where(cnt > 0.0, cnt, 1.0)
    d0, d1, d2 = s0 / safe, s1 / safe, s2 / safe
    pt = d0 * d0 + d1 * d1 + d2 * d2
    kidx = lax.broadcasted_iota(jnp.int32, (1, KP), 1)
    valid = kidx > 0
    contrib = (cnt >= 2.0) & valid
    present = (cnt >= 1.0) & valid
    total = jnp.sum(jnp.where(contrib, pt, 0.0))
    ntree = jnp.sum(jnp.where(present, 1.0, 0.0))
    loss = jnp.where(ntree > 0.0, total / jnp.maximum(ntree, 1.0), 0.0)
    o_ref[...] = jnp.full((1, 1), loss, jnp.float32)


def kernel(coords, offset_inst, offset_tree, tree_labels):
    del coords
    dx = offset_inst[:, 0] - offset_tree[:, 0]
    dy = offset_inst[:, 1] - offset_tree[:, 1]
    dz = offset_inst[:, 2] - offset_tree[:, 2]
    partials = _sc_segment_sums(dx, dy, dz, tree_labels)
    loss = pl.pallas_call(
        _loss_body,
        out_shape=jax.ShapeDtypeStruct((1, 1), jnp.float32),
    )(partials)
    return jnp.reshape(loss, ())
